# baseline (speedup 1.0000x reference)
import jax
import jax.numpy as jnp
from jax.experimental import pallas as pl
from jax.experimental.pallas import tpu as pltpu

_B, _C1, _C2, _H, _W = 4, 384, 384, 56, 56
_E, _K = 8, 2
_HW = _H * _W

_NT = 2
_C2T = _C2 // _NT


def _moe_kernel(x_ref, Wr_ref, We_ref, Ws_ref, out_ref, Wc_ref):
    nt = pl.program_id(1)

    @pl.when(nt == 0)
    def _route():
        gap = jnp.mean(x_ref[0], axis=1, keepdims=True)
        logits = jnp.sum(gap * Wr_ref[...], axis=0, keepdims=True)
        iota = jax.lax.broadcasted_iota(jnp.int32, (1, _E), 1)
        m1 = jnp.max(logits)
        i1 = jnp.min(jnp.where(logits == m1, iota, _E))
        masked = jnp.where(iota == i1, -jnp.inf, logits)
        m2 = jnp.max(masked)
        i2 = jnp.min(jnp.where(masked == m2, iota, _E))
        e = jnp.exp(m2 - m1)
        w0 = 1.0 / (1.0 + e)
        w1 = e / (1.0 + e)
        Wc_ref[...] = (w0 * We_ref[i1] + w1 * We_ref[i2] + Ws_ref[...]).astype(
            jnp.bfloat16
        )

    rows = pl.ds(nt * _C2T, _C2T)
    xb = x_ref[0].astype(jnp.bfloat16)
    out_ref[0] = (
        jnp.dot(Wc_ref[rows, :], xb, preferred_element_type=jnp.float32)
        + x_ref[0, rows, :]
    )


def kernel(x, Wr, We, Ws):
    xr = x.reshape(_B, _C1, _HW)
    out = pl.pallas_call(
        _moe_kernel,
        grid=(_B, _NT),
        in_specs=[
            pl.BlockSpec((1, _C1, _HW), lambda b, nt: (b, 0, 0)),
            pl.BlockSpec((_C1, _E), lambda b, nt: (0, 0)),
            pl.BlockSpec((_E, _C2, _C1), lambda b, nt: (0, 0, 0)),
            pl.BlockSpec((_C2, _C1), lambda b, nt: (0, 0)),
        ],
        out_specs=pl.BlockSpec((1, _C2T, _HW), lambda b, nt: (b, nt, 0)),
        out_shape=jax.ShapeDtypeStruct((_B, _C2, _HW), jnp.float32),
        scratch_shapes=[pltpu.VMEM((_C2, _C1), jnp.bfloat16)],
        compiler_params=pltpu.CompilerParams(
            dimension_semantics=("arbitrary", "arbitrary"),
            vmem_limit_bytes=100 * 1024 * 1024,
        ),
    )(xr, Wr, We, Ws)
    return out.reshape(_B, _C2, _H, _W)


# speedup vs baseline: 1.1635x; 1.1635x over previous
import jax
import jax.numpy as jnp
from jax.experimental import pallas as pl
from jax.experimental.pallas import tpu as pltpu

_B, _C1, _C2, _H, _W = 4, 384, 384, 56, 56
_E, _K = 8, 2
_HW = _H * _W


def _moe_kernel(x_ref, Wr_ref, We_ref, Ws_ref, out_ref):
    xb = x_ref[0]
    gap = jnp.mean(xb, axis=1, keepdims=True)
    logits = jnp.sum(gap * Wr_ref[...], axis=0, keepdims=True)
    iota = jax.lax.broadcasted_iota(jnp.int32, (1, _E), 1)
    m1 = jnp.max(logits)
    i1 = jnp.min(jnp.where(logits == m1, iota, _E))
    masked = jnp.where(iota == i1, -jnp.inf, logits)
    m2 = jnp.max(masked)
    i2 = jnp.min(jnp.where(masked == m2, iota, _E))
    e = jnp.exp(m2 - m1)
    w0 = 1.0 / (1.0 + e)
    w1 = e / (1.0 + e)
    Wc = w0 * We_ref[i1] + w1 * We_ref[i2] + Ws_ref[...]
    out_ref[0] = jnp.dot(Wc, xb, preferred_element_type=jnp.float32) + xb


def kernel(x, Wr, We, Ws):
    xr = x.reshape(_B, _C1, _HW)
    out = pl.pallas_call(
        _moe_kernel,
        grid=(_B,),
        in_specs=[
            pl.BlockSpec((1, _C1, _HW), lambda b: (b, 0, 0)),
            pl.BlockSpec((_C1, _E), lambda b: (0, 0)),
            pl.BlockSpec((_E, _C2, _C1), lambda b: (0, 0, 0)),
            pl.BlockSpec((_C2, _C1), lambda b: (0, 0)),
        ],
        out_specs=pl.BlockSpec((1, _C2, _HW), lambda b: (b, 0, 0)),
        out_shape=jax.ShapeDtypeStruct((_B, _C2, _HW), jnp.float32),
        compiler_params=pltpu.CompilerParams(
            dimension_semantics=("arbitrary",),
        ),
    )(xr, Wr, We, Ws)
    return out.reshape(_B, _C2, _H, _W)


# speedup vs baseline: 1.1771x; 1.0117x over previous
import jax
import jax.numpy as jnp
from jax.experimental import pallas as pl
from jax.experimental.pallas import tpu as pltpu

_B, _C1, _C2, _H, _W = 4, 384, 384, 56, 56
_E, _K = 8, 2
_HW = _H * _W
_BS = 2


def _moe_kernel(x_ref, Wr_ref, We_ref, Ws_ref, out_ref):
    for s in range(_BS):
        xb = x_ref[s]
        gap = jnp.mean(xb, axis=1, keepdims=True)
        logits = jnp.sum(gap * Wr_ref[...], axis=0, keepdims=True)
        iota = jax.lax.broadcasted_iota(jnp.int32, (1, _E), 1)
        m1 = jnp.max(logits)
        i1 = jnp.min(jnp.where(logits == m1, iota, _E))
        masked = jnp.where(iota == i1, -jnp.inf, logits)
        m2 = jnp.max(masked)
        i2 = jnp.min(jnp.where(masked == m2, iota, _E))
        e = jnp.exp(m2 - m1)
        w0 = 1.0 / (1.0 + e)
        w1 = e / (1.0 + e)
        Wc = w0 * We_ref[i1] + w1 * We_ref[i2] + Ws_ref[...]
        out_ref[s] = jnp.dot(Wc, xb, preferred_element_type=jnp.float32) + xb


def kernel(x, Wr, We, Ws):
    xr = x.reshape(_B, _C1, _HW)
    out = pl.pallas_call(
        _moe_kernel,
        grid=(_B // _BS,),
        in_specs=[
            pl.BlockSpec((_BS, _C1, _HW), lambda b: (b, 0, 0)),
            pl.BlockSpec((_C1, _E), lambda b: (0, 0)),
            pl.BlockSpec((_E, _C2, _C1), lambda b: (0, 0, 0)),
            pl.BlockSpec((_C2, _C1), lambda b: (0, 0)),
        ],
        out_specs=pl.BlockSpec((_BS, _C2, _HW), lambda b: (b, 0, 0)),
        out_shape=jax.ShapeDtypeStruct((_B, _C2, _HW), jnp.float32),
        compiler_params=pltpu.CompilerParams(
            dimension_semantics=("arbitrary",),
            vmem_limit_bytes=100 * 1024 * 1024,
        ),
    )(xr, Wr, We, Ws)
    return out.reshape(_B, _C2, _H, _W)
